# Initial kernel scaffold; baseline (speedup 1.0000x reference)
import jax
import jax.numpy as jnp
from jax.experimental import pallas as pl


def kernel(x, edge_index, edge_order, graph_ids, W_emb, b_emb, lin0, mW0, mb0, g0, be0, lin1, mW1, mb1, g1, be1, cW1, cb1, cW2, cb2):
    raise NotImplementedError("write your pallas kernel here")


# speedup vs baseline: 4.2105x; 4.2105x over previous
import functools

import jax
import jax.numpy as jnp
from jax import lax
from jax.experimental import pallas as pl
from jax.experimental.pallas import tpu as pltpu
from jax.experimental.pallas import tpu_sc as plsc

_N = 10000
_E = 320000
_HID = 128
_HALF = 64
_K = 3
_G = 16

_NC = 2
_NS = 16

_HALF_ROWS = (_K * _N) // _NC
_ACC_ROWS = 15024
_CP_STEP = 936
_CP_LEN = 984
_EDGES_PER_TILE = _E // _NS
_CHUNK = 80
_ITERS = _EDGES_PER_TILE // _CHUNK


def _sc_scatter_body(h_hbm, src_hbm, dst_hbm, ord_hbm, zeros_hbm, out_hbm,
                     src_v, dst_v, ord_v, sidx_v, rows_v, acc, sem):
    c = lax.axis_index("c")
    s = lax.axis_index("s")
    pltpu.sync_copy(zeros_hbm, acc.at[pl.ds(s * _CP_STEP, _CP_LEN)])
    plsc.subcore_barrier()

    base = s * _EDGES_PER_TILE
    row_lo = c * _HALF_ROWS
    trash = _HALF_ROWS + s

    def body(i, carry):
        off = base + i * _CHUNK
        cp_s = pltpu.async_copy(src_hbm.at[pl.ds(off, _CHUNK)], src_v, sem)
        cp_d = pltpu.async_copy(dst_hbm.at[pl.ds(off, _CHUNK)], dst_v, sem)
        cp_o = pltpu.async_copy(ord_hbm.at[pl.ds(off, _CHUNK)], ord_v, sem)
        cp_s.wait()
        cp_d.wait()
        cp_o.wait()
        for j in range(_CHUNK // 16):
            sl = pl.ds(j * 16, 16)
            local = ord_v[sl] * _N + dst_v[sl] - row_lo
            ok = (local >= 0) & (local < _HALF_ROWS)
            sidx_v[sl] = jnp.where(ok, local, trash)
        pltpu.async_copy(h_hbm.at[src_v], rows_v, sem).wait()
        pltpu.sync_copy(rows_v, acc.at[sidx_v], add=True)
        return carry

    lax.fori_loop(0, _ITERS, body, 0)
    plsc.subcore_barrier()
    pltpu.sync_copy(acc.at[pl.ds(s * _CP_STEP, _CP_LEN)],
                    out_hbm.at[c, pl.ds(s * _CP_STEP, _CP_LEN)])


@functools.cache
def _sc_scatter():
  return pl.kernel(
    _sc_scatter_body,
    out_type=jax.ShapeDtypeStruct((_NC, _ACC_ROWS, _HID), jnp.float32),
    mesh=plsc.VectorSubcoreMesh(core_axis_name="c", subcore_axis_name="s",
                                num_cores=_NC, num_subcores=_NS),
    scratch_types=[
        pltpu.VMEM((_CHUNK,), jnp.int32),
        pltpu.VMEM((_CHUNK,), jnp.int32),
        pltpu.VMEM((_CHUNK,), jnp.int32),
        pltpu.VMEM((_CHUNK,), jnp.int32),
        pltpu.VMEM((_CHUNK, _HID), jnp.float32),
        pltpu.VMEM_SHARED((_ACC_ROWS, _HID), jnp.float32),
        pltpu.SemaphoreType.DMA,
    ],
  )


def _r(a):
    return a.astype(jnp.bfloat16).astype(jnp.float32)


def _embed_body(x_ref, w_ref, b_ref, out_ref):
    h = jnp.dot(_r(x_ref[...]), _r(w_ref[...]),
                preferred_element_type=jnp.float32,
                precision=lax.Precision.HIGHEST) + b_ref[...]
    out_ref[...] = _r(h)


_embed = pl.pallas_call(
    _embed_body,
    out_shape=jax.ShapeDtypeStruct((_N, _HID), jnp.float32),
)


def _agg_bn(s_ref, lin_ref, mw_ref, mb_ref, g_ref, be_ref):
    big0 = s_ref[0, pl.ds(0, _HALF_ROWS), :]
    big1 = s_ref[1, pl.ds(0, _HALF_ROWS), :]
    big = jnp.concatenate([big0, big1], axis=0)
    agg = jnp.zeros((_N, _HID), dtype=jnp.float32)
    for k in range(_K):
        agg = agg + jnp.dot(big[k * _N:(k + 1) * _N, :], _r(lin_ref[k]),
                            preferred_element_type=jnp.float32, precision=lax.Precision.HIGHEST)
    h2 = jnp.maximum(
        jnp.dot(_r(agg), _r(mw_ref[...]), preferred_element_type=jnp.float32, precision=lax.Precision.HIGHEST)
        + mb_ref[...], 0.0)
    mean = jnp.mean(h2, axis=0, keepdims=True)
    var = jnp.mean((h2 - mean) * (h2 - mean), axis=0, keepdims=True)
    return g_ref[...] * (h2 - mean) / jnp.sqrt(var + 1e-5) + be_ref[...]


def _dense_mid_body(s_ref, lin_ref, mw_ref, mb_ref, g_ref, be_ref, out_ref):
    out_ref[...] = _r(_agg_bn(s_ref, lin_ref, mw_ref, mb_ref, g_ref, be_ref))


_dense_mid = pl.pallas_call(
    _dense_mid_body,
    out_shape=jax.ShapeDtypeStruct((_N, _HID), jnp.float32),
)


def _dense_final_body(s_ref, lin_ref, mw_ref, mb_ref, g_ref, be_ref,
                      gid_ref, cw1_ref, cb1_ref, cw2_ref, cb2_ref, out_ref):
    h = _agg_bn(s_ref, lin_ref, mw_ref, mb_ref, g_ref, be_ref)
    gids = gid_ref[...]
    rows = lax.broadcasted_iota(jnp.int32, (_G, _N), 0)
    onehot = jnp.where(rows == gids, 1.0, 0.0).astype(jnp.float32)
    pooled = jnp.dot(onehot, h, preferred_element_type=jnp.float32, precision=lax.Precision.HIGHEST)
    hid = jnp.maximum(
        jnp.dot(_r(pooled), _r(cw1_ref[...]), preferred_element_type=jnp.float32, precision=lax.Precision.HIGHEST)
        + cb1_ref[...], 0.0)
    out_ref[...] = (jnp.dot(_r(hid), _r(cw2_ref[...]),
                            preferred_element_type=jnp.float32, precision=lax.Precision.HIGHEST) + cb2_ref[...])


_dense_final = pl.pallas_call(
    _dense_final_body,
    out_shape=jax.ShapeDtypeStruct((_G, 10), jnp.float32),
)


@jax.jit
def kernel(x, edge_index, edge_order, graph_ids, W_emb, b_emb,
           lin0, mW0, mb0, g0, be0,
           lin1, mW1, mb1, g1, be1,
           cW1, cb1, cW2, cb2):
    src = edge_index[0]
    dst = edge_index[1]
    zeros = jnp.zeros((_CP_LEN, _HID), dtype=jnp.float32)

    sc_scatter = _sc_scatter()
    h = _embed(x, W_emb, b_emb.reshape(1, _HID))
    s0 = sc_scatter(h, src, dst, edge_order, zeros)
    h = _dense_mid(s0, lin0, mW0, mb0.reshape(1, _HID),
                   g0.reshape(1, _HID), be0.reshape(1, _HID))
    s1 = sc_scatter(h, src, dst, edge_order, zeros)
    logits = _dense_final(s1, lin1, mW1, mb1.reshape(1, _HID),
                          g1.reshape(1, _HID), be1.reshape(1, _HID),
                          graph_ids.reshape(1, _N).astype(jnp.int32),
                          cW1, cb1.reshape(1, _HALF), cW2, cb2.reshape(1, 10))
    return logits
